# baseline (speedup 1.0000x reference)
import functools
import math

import jax
import jax.numpy as jnp
from jax import lax
from jax.experimental import pallas as pl
from jax.experimental.pallas import tpu as pltpu
from jax.experimental.pallas import tpu_sc as plsc

_D = 64
_SCALE = math.sqrt(_D)


def _embed(idx, table_pairs):
    (B,) = idx.shape
    info = plsc.get_sparse_core_info()
    nw = info.num_cores * info.num_subcores
    b_per_w = B // nw
    chunk = 128
    n_chunks = b_per_w // chunk

    mesh = plsc.VectorSubcoreMesh(core_axis_name="c", subcore_axis_name="s")

    @functools.partial(
        pl.kernel,
        out_type=jax.ShapeDtypeStruct((B, _D), jnp.float32),
        mesh=mesh,
        scratch_types=[
            pltpu.VMEM((b_per_w,), jnp.int32),
            pltpu.VMEM((chunk,), jnp.int32),
            pltpu.VMEM((chunk,), jnp.int32),
            pltpu.VMEM((chunk, 2 * _D), jnp.float32),
            pltpu.VMEM((chunk, 2 * _D), jnp.float32),
            pltpu.VMEM((chunk, _D), jnp.float32),
            pltpu.VMEM((chunk, _D), jnp.float32),
            pltpu.SemaphoreType.DMA,
            pltpu.SemaphoreType.DMA,
            pltpu.SemaphoreType.DMA,
            pltpu.SemaphoreType.DMA,
        ],
        compiler_params=pltpu.CompilerParams(use_tc_tiling_on_sc=True),
    )
    def emb(idx_hbm, table_hbm, out_hbm, idx_v, jbuf0, jbuf1, rows0, rows1,
            stage0, stage1, gsem0, gsem1, ssem0, ssem1):
        wid = lax.axis_index("s") * info.num_cores + lax.axis_index("c")
        base = wid * b_per_w
        bufs = ((rows0, stage0, jbuf0, gsem0, ssem0),
                (rows1, stage1, jbuf1, gsem1, ssem1))

        def gather(jbuf, rows, sem):
            return pltpu.make_async_copy(table_hbm.at[jbuf], rows, sem)

        def store(c, stage, sem):
            return pltpu.make_async_copy(
                stage, out_hbm.at[pl.ds(base + c * chunk, chunk)], sem)

        def fill_jbuf(jbuf, c):
            @pl.loop(0, chunk, step=16)
            def _(v):
                jbuf[pl.ds(v, 16)] = lax.shift_right_logical(
                    idx_v[pl.ds(c * chunk + v, 16)], 1)

        pltpu.sync_copy(idx_hbm.at[pl.ds(base, b_per_w)], idx_v)
        fill_jbuf(jbuf0, 0)
        gather(jbuf0, rows0, gsem0).start()

        @pl.loop(0, n_chunks, step=2)
        def _(ci):
            for b in range(2):
                cur = ci + b
                rows, stage, jbuf, gsem, ssem = bufs[b]
                nrows, nstage, njbuf, ngsem, nssem = bufs[1 - b]
                nxt = cur + 1

                @pl.when(nxt < n_chunks)
                def _():
                    @pl.when(nxt >= 2)
                    def _():
                        store(nxt - 2, nstage, nssem).wait()

                    fill_jbuf(njbuf, nxt)
                    gather(njbuf, nrows, ngsem).start()

                gather(jbuf, rows, gsem).wait()

                def pick_scale(g, c2):
                    r0 = g * 16
                    offv = (idx_v[pl.ds(cur * chunk + r0, 16)] & 1) * _D
                    for r in range(16):
                        off = offv[r]
                        for j in range(_D // 16):
                            stage[r0 + r, pl.ds(j * 16, 16)] = (
                                rows[r0 + r, pl.ds(off + j * 16, 16)]
                                * _SCALE)
                    return c2

                lax.fori_loop(0, chunk // 16, pick_scale, 0)
                store(cur, stage, ssem).start()

        store(n_chunks - 2, stage0, ssem0).wait()
        store(n_chunks - 1, stage1, ssem1).wait()

    return emb(idx, table_pairs)


def kernel(input, table):
    b, s = input.shape
    idx = input.reshape(b * s).astype(jnp.int32)
    pairs = table.reshape(table.shape[0] // 2, 2 * _D)
    out = _embed(idx, pairs)
    return out.reshape(b, s, _D)
